# baseline (speedup 1.0000x reference)
import jax
import jax.numpy as jnp
from jax.experimental import pallas as pl
from jax.experimental.pallas import tpu as pltpu

_N_STREAMS = 2


def _sigmoid(x):
    return 0.5 + 0.5 * jnp.tanh(0.5 * x)


def _gru_chunk_kernel(x_ref, wx_ref, b_ref, wh_zr_ref, wh_n_ref,
                      hist_ref, h_carry):
    b2, t_tile, d = x_ref.shape
    hidden = wh_n_ref.shape[1]
    bs = b2 // _N_STREAMS

    @pl.when(pl.program_id(0) == 0)
    def _():
        h_carry[...] = jnp.zeros_like(h_carry)

    x_t = x_ref[...].transpose(1, 0, 2)
    x2d = x_t.reshape(t_tile * b2, d).astype(jnp.bfloat16)
    xp = (jnp.dot(x2d, wx_ref[...], preferred_element_type=jnp.float32)
          + b_ref[...])
    xp_zr = xp[:, :2 * hidden].reshape(t_tile, b2, 2 * hidden)
    xp_n = xp[:, 2 * hidden:].reshape(t_tile, b2, hidden)

    wh_zr = wh_zr_ref[...]
    wh_n = wh_n_ref[...]

    hs = [h_carry[s * bs:(s + 1) * bs, :] for s in range(_N_STREAMS)]
    for t in range(t_tile):
        azr = [jnp.dot(hs[s].astype(jnp.bfloat16), wh_zr,
                       preferred_element_type=jnp.float32)
               + xp_zr[t, s * bs:(s + 1) * bs, :]
               for s in range(_N_STREAMS)]
        zr = [_sigmoid(a) for a in azr]
        rh = [zr[s][:, hidden:] * hs[s] for s in range(_N_STREAMS)]
        an = [jnp.dot(rh[s].astype(jnp.bfloat16), wh_n,
                      preferred_element_type=jnp.float32)
              + xp_n[t, s * bs:(s + 1) * bs, :]
              for s in range(_N_STREAMS)]
        for s in range(_N_STREAMS):
            n = jnp.tanh(an[s])
            z = zr[s][:, :hidden]
            hs[s] = hs[s] + z * (n - hs[s])
            hist_ref[t, s * bs:(s + 1) * bs, :] = hs[s]

    for s in range(_N_STREAMS):
        h_carry[s * bs:(s + 1) * bs, :] = hs[s]


def _largest_divisor_leq(n, cap):
    for cand in range(min(n, cap), 0, -1):
        if n % cand == 0:
            return cand
    return 1


@jax.jit
def kernel(x_btd, wz, bz, wr, br, wn, bn):
    B, T, D = x_btd.shape
    H = wz.shape[1]
    if T == 0:
        return jnp.zeros((0, B, H), jnp.float32)

    wx = jnp.concatenate([wz[H:], wr[H:], wn[H:]], axis=1)
    b = jnp.concatenate([bz, br, bn], axis=1)
    wh_zr = jnp.concatenate([wz[:H], wr[:H]], axis=1)
    wh_n = wn[:H]
    wx = wx.astype(jnp.bfloat16)
    wh_zr = wh_zr.astype(jnp.bfloat16)
    wh_n = wh_n.astype(jnp.bfloat16)

    nb = 2
    B2 = B // nb
    t_tile = _largest_divisor_leq(T, 8)
    grid = (T // t_tile, nb)

    return pl.pallas_call(
        _gru_chunk_kernel,
        out_shape=jax.ShapeDtypeStruct((T, B, H), jnp.float32),
        grid=grid,
        in_specs=[
            pl.BlockSpec((B2, t_tile, D), lambda i, j: (j, i, 0)),
            pl.BlockSpec((D, 3 * H), lambda i, j: (0, 0)),
            pl.BlockSpec((1, 3 * H), lambda i, j: (0, 0)),
            pl.BlockSpec((H, 2 * H), lambda i, j: (0, 0)),
            pl.BlockSpec((H, H), lambda i, j: (0, 0)),
        ],
        out_specs=pl.BlockSpec((t_tile, B2, H), lambda i, j: (i, j, 0)),
        scratch_shapes=[pltpu.VMEM((B2, H), jnp.float32)],
        compiler_params=pltpu.CompilerParams(
            dimension_semantics=("arbitrary", "parallel")),
    )(x_btd, wx, b, wh_zr, wh_n)


# speedup vs baseline: 1.5577x; 1.5577x over previous
import jax
import jax.numpy as jnp
from jax.experimental import pallas as pl
from jax.experimental.pallas import tpu as pltpu

_N_STREAMS = 2


def _sigmoid(x):
    return 0.5 + 0.5 * jnp.tanh(0.5 * x)


def _gru_chunk_kernel(x_ref, wx_ref, b_ref, wh_zr_ref, wh_n_ref,
                      hist_ref, h_carry):
    b2, t_tile, d = x_ref.shape
    hidden = wh_n_ref.shape[1]
    bs = b2 // _N_STREAMS

    @pl.when(pl.program_id(1) == 0)
    def _():
        h_carry[...] = jnp.zeros_like(h_carry)

    x_t = x_ref[...].astype(jnp.bfloat16).transpose(1, 0, 2)
    x2d = x_t.reshape(t_tile * b2, d)
    xp = (jnp.dot(x2d, wx_ref[...], preferred_element_type=jnp.float32)
          + b_ref[...])
    xp_zr = xp[:, :2 * hidden].reshape(t_tile, b2, 2 * hidden)
    xp_n = xp[:, 2 * hidden:].reshape(t_tile, b2, hidden)

    wh_zr = wh_zr_ref[...]
    wh_n = wh_n_ref[...]

    hs = [h_carry[s * bs:(s + 1) * bs, :] for s in range(_N_STREAMS)]
    for t in range(t_tile):
        azr = [jnp.dot(hs[s].astype(jnp.bfloat16), wh_zr,
                       preferred_element_type=jnp.float32)
               + xp_zr[t, s * bs:(s + 1) * bs, :]
               for s in range(_N_STREAMS)]
        zr = [_sigmoid(a) for a in azr]
        rh = [zr[s][:, hidden:] * hs[s] for s in range(_N_STREAMS)]
        an = [jnp.dot(rh[s].astype(jnp.bfloat16), wh_n,
                      preferred_element_type=jnp.float32)
              + xp_n[t, s * bs:(s + 1) * bs, :]
              for s in range(_N_STREAMS)]
        for s in range(_N_STREAMS):
            n = jnp.tanh(an[s])
            z = zr[s][:, :hidden]
            hs[s] = hs[s] + z * (n - hs[s])
            hist_ref[t, s * bs:(s + 1) * bs, :] = hs[s]

    for s in range(_N_STREAMS):
        h_carry[s * bs:(s + 1) * bs, :] = hs[s]


def _largest_divisor_leq(n, cap):
    for cand in range(min(n, cap), 0, -1):
        if n % cand == 0:
            return cand
    return 1


@jax.jit
def kernel(x_btd, wz, bz, wr, br, wn, bn):
    B, T, D = x_btd.shape
    H = wz.shape[1]
    if T == 0:
        return jnp.zeros((0, B, H), jnp.float32)

    wx = jnp.concatenate([wz[H:], wr[H:], wn[H:]], axis=1)
    b = jnp.concatenate([bz, br, bn], axis=1)
    wh_zr = jnp.concatenate([wz[:H], wr[:H]], axis=1)
    wh_n = wn[:H]
    wx = wx.astype(jnp.bfloat16)
    wh_zr = wh_zr.astype(jnp.bfloat16)
    wh_n = wh_n.astype(jnp.bfloat16)

    nb = 1
    B2 = B // nb
    t_tile = _largest_divisor_leq(T, 16)
    grid = (nb, T // t_tile)

    return pl.pallas_call(
        _gru_chunk_kernel,
        out_shape=jax.ShapeDtypeStruct((T, B, H), jnp.float32),
        grid=grid,
        in_specs=[
            pl.BlockSpec((B2, t_tile, D), lambda j, i: (j, i, 0)),
            pl.BlockSpec((D, 3 * H), lambda j, i: (0, 0)),
            pl.BlockSpec((1, 3 * H), lambda j, i: (0, 0)),
            pl.BlockSpec((H, 2 * H), lambda j, i: (0, 0)),
            pl.BlockSpec((H, H), lambda j, i: (0, 0)),
        ],
        out_specs=pl.BlockSpec((t_tile, B2, H), lambda j, i: (i, j, 0)),
        scratch_shapes=[pltpu.VMEM((B2, H), jnp.float32)],
        compiler_params=pltpu.CompilerParams(
            dimension_semantics=("arbitrary", "arbitrary")),
    )(x_btd, wx, b, wh_zr, wh_n)
